# Initial kernel scaffold; baseline (speedup 1.0000x reference)
import functools

import jax
import jax.numpy as jnp
from jax import lax
from jax.experimental import pallas as pl
from jax.experimental.pallas import tpu as pltpu
from jax.experimental.pallas import tpu_sc as plsc

NN = 10000
NROW = 80
EE = 320000
EPAIR = 320000
NC, NS = 2, 16
NW = NC * NS


def _mesh():
    return plsc.VectorSubcoreMesh(core_axis_name="c", subcore_axis_name="s")


EPT_DEG = EE // NS


def _deg_body(dst_hbm, out_hbm, dstv, degl, idxv, acc):
    cid = lax.axis_index("c")
    sid = lax.axis_index("s")

    @pl.when(cid == 0)
    def _():
        pltpu.sync_copy(dst_hbm.at[pl.ds(sid * EPT_DEG, EPT_DEG)], dstv)

        def zero_row(i, carry):
            degl[lax.shift_right_logical(i, 3),
                 pl.ds(lax.mul(lax.rem(i, 8), 16), 16)] = jnp.zeros(
                     (16,), jnp.float32)
            return carry

        lax.fori_loop(0, NROW * 8, zero_row, 0)
        pltpu.sync_copy(degl.at[pl.ds(sid * 5, 5)], acc.at[pl.ds(sid * 5, 5)])

        def fill_idx(i, carry):
            idxv[pl.ds(i * 16, 16)] = lax.iota(jnp.int32, 16) + i * 16
            return carry

        lax.fori_loop(0, NROW // 16, fill_idx, 0)

        ones = jnp.ones((16,), jnp.float32)

        def edge_step(i, carry):
            d = dstv[pl.ds(i * 16, 16)]
            plsc.addupdate_scatter(
                degl,
                [lax.shift_right_logical(d, 7), lax.bitwise_and(d, 127)],
                ones)
            return carry

        lax.fori_loop(0, EPT_DEG // 16, edge_step, 0)
        plsc.subcore_barrier()
        pltpu.sync_copy(degl, acc.at[idxv], add=True)
        plsc.subcore_barrier()
        pltpu.sync_copy(acc.at[pl.ds(sid * 5, 5)],
                        out_hbm.at[pl.ds(sid * 5, 5)])


_deg_kernel = pl.kernel(
    _deg_body,
    out_type=jax.ShapeDtypeStruct((NROW, 128), jnp.float32),
    mesh=_mesh(),
    scratch_types=[
        pltpu.VMEM((EPT_DEG,), jnp.int32),
        pltpu.VMEM((NROW, 128), jnp.float32),
        pltpu.VMEM((NROW,), jnp.int32),
        pltpu.VMEM_SHARED((NROW, 128), jnp.float32),
    ],
)


EPT = EE // NW
KCH = 80
NCH = EPT // KCH
RPT = NN // NS
ZR = 125


def _make_spmm(C):
    cb = C // 16

    def body(g_hbm, src_hbm, dst_hbm, out_a, out_b,
             sidx, didx, rows, zbuf, acc, sem):
        cid = lax.axis_index("c")
        sid = lax.axis_index("s")
        wid = sid * NC + cid
        base = wid * EPT

        def zero_row(i, carry):
            zbuf[lax.div(i, cb), pl.ds(lax.mul(lax.rem(i, cb), 16), 16)] = (
                jnp.zeros((16,), jnp.float32))
            return carry

        lax.fori_loop(0, ZR * cb, zero_row, 0)
        for z in range(RPT // ZR):
            pltpu.sync_copy(zbuf, acc.at[pl.ds(sid * RPT + z * ZR, ZR)])
        plsc.subcore_barrier()

        def edge_step(i, carry):
            off = base + i * KCH
            pltpu.sync_copy(src_hbm.at[pl.ds(off, KCH)], sidx)
            pltpu.sync_copy(dst_hbm.at[pl.ds(off, KCH)], didx)
            pltpu.async_copy(g_hbm.at[sidx], rows, sem).wait()
            pltpu.sync_copy(rows, acc.at[didx], add=True)
            return carry

        lax.fori_loop(0, NCH, edge_step, 0)
        plsc.subcore_barrier()

        @pl.when(cid == 0)
        def _():
            pltpu.sync_copy(acc.at[pl.ds(sid * RPT, RPT)],
                            out_a.at[pl.ds(sid * RPT, RPT)])

        @pl.when(cid == 1)
        def _():
            pltpu.sync_copy(acc.at[pl.ds(sid * RPT, RPT)],
                            out_b.at[pl.ds(sid * RPT, RPT)])

    return pl.kernel(
        body,
        out_type=[jax.ShapeDtypeStruct((NN, C), jnp.float32)] * 2,
        mesh=_mesh(),
        scratch_types=[
            pltpu.VMEM((KCH,), jnp.int32),
            pltpu.VMEM((KCH,), jnp.int32),
            pltpu.VMEM((KCH, C), jnp.float32),
            pltpu.VMEM((ZR, C), jnp.float32),
            pltpu.VMEM_SHARED((NN, C), jnp.float32),
            pltpu.SemaphoreType.DMA,
        ],
    )


_spmm128 = _make_spmm(128)
_spmm64 = _make_spmm(64)


EPT_F = EPAIR // NW


def _final_body(s1_hbm, s2_hbm, src_hbm, dst_hbm, out_hbm,
                s1v, s2v, siv, dov, ov):
    cid = lax.axis_index("c")
    sid = lax.axis_index("s")
    wid = sid * NC + cid
    base = wid * EPT_F
    pltpu.sync_copy(s1_hbm, s1v)
    pltpu.sync_copy(s2_hbm, s2v)
    pltpu.sync_copy(src_hbm.at[pl.ds(base, EPT_F)], siv)
    pltpu.sync_copy(dst_hbm.at[pl.ds(base, EPT_F)], dov)

    def edge_step(i, carry):
        s = siv[pl.ds(i * 16, 16)]
        d = dov[pl.ds(i * 16, 16)]
        ov[pl.ds(i * 16, 16)] = (plsc.load_gather(s1v, [s]) +
                                 plsc.load_gather(s2v, [d]))
        return carry

    lax.fori_loop(0, EPT_F // 16, edge_step, 0)
    pltpu.sync_copy(ov, out_hbm.at[pl.ds(base, EPT_F)])


_final_kernel = pl.kernel(
    _final_body,
    out_type=jax.ShapeDtypeStruct((EPAIR,), jnp.float32),
    mesh=_mesh(),
    scratch_types=[
        pltpu.VMEM((NN,), jnp.float32),
        pltpu.VMEM((NN,), jnp.float32),
        pltpu.VMEM((EPT_F,), jnp.int32),
        pltpu.VMEM((EPT_F,), jnp.int32),
        pltpu.VMEM((EPT_F,), jnp.float32),
    ],
)


BLK = 1250
GRID = NN // BLK


def _tc_a_body(x_ref, w_ref, deg_ref, g1_ref):
    dinv = lax.rsqrt(deg_ref[...] + 1.0)
    g1_ref[...] = dinv * jnp.dot(x_ref[...], w_ref[...],
                                 preferred_element_type=jnp.float32)


def _tc_b_body(ta, tb, g1, deg, b1, w2, g2):
    dinv = lax.rsqrt(deg[...] + 1.0)
    h1 = jnp.maximum(dinv * (ta[...] + tb[...] + g1[...]) + b1[...], 0.0)
    g2[...] = dinv * jnp.dot(h1, w2[...], preferred_element_type=jnp.float32)


def _tc_c_body(ta, tb, g2, deg, b2, fw1, fw2, fb, s1, s2):
    dinv = lax.rsqrt(deg[...] + 1.0)
    h2 = jnp.maximum(dinv * (ta[...] + tb[...] + g2[...]) + b2[...], 0.0)
    s1[...] = jnp.dot(h2, fw1[...], preferred_element_type=jnp.float32) + fb[...]
    s2[...] = jnp.dot(h2, fw2[...], preferred_element_type=jnp.float32)


def _row_spec(c):
    return pl.BlockSpec((BLK, c), lambda i: (i, 0))


def _full_spec(r, c):
    return pl.BlockSpec((r, c), lambda i: (0, 0))


_tc_a = pl.pallas_call(
    _tc_a_body,
    grid=(GRID,),
    in_specs=[_row_spec(128), _full_spec(128, 128), _row_spec(1)],
    out_specs=_row_spec(128),
    out_shape=jax.ShapeDtypeStruct((NN, 128), jnp.float32),
)

_tc_b = pl.pallas_call(
    _tc_b_body,
    grid=(GRID,),
    in_specs=[_row_spec(128), _row_spec(128), _row_spec(128), _row_spec(1),
              _full_spec(1, 128), _full_spec(128, 64)],
    out_specs=_row_spec(64),
    out_shape=jax.ShapeDtypeStruct((NN, 64), jnp.float32),
)

_tc_c = pl.pallas_call(
    _tc_c_body,
    grid=(GRID,),
    in_specs=[_row_spec(64), _row_spec(64), _row_spec(64), _row_spec(1),
              _full_spec(1, 64), _full_spec(64, 1), _full_spec(64, 1),
              _full_spec(1, 1)],
    out_specs=[_row_spec(1), _row_spec(1)],
    out_shape=[jax.ShapeDtypeStruct((NN, 1), jnp.float32)] * 2,
)


def kernel(x, edge_index, edge_index_pos, edge_index_neg,
           W1, b1, W2, b2, fc_w, fc_b):
    src = edge_index[0]
    dst = edge_index[1]

    deg2d = _deg_kernel(dst)
    deg_col = deg2d.reshape(-1)[:NN].reshape(NN, 1)

    g1 = _tc_a(x, W1, deg_col)
    t1a, t1b = _spmm128(g1, src, dst)
    g2 = _tc_b(t1a, t1b, g1, deg_col, b1.reshape(1, 128), W2)
    t2a, t2b = _spmm64(g2, src, dst)
    s1, s2 = _tc_c(t2a, t2b, g2, deg_col, b2.reshape(1, 64),
                   fc_w[:64], fc_w[64:], fc_b.reshape(1, 1))

    esrc = jnp.concatenate([edge_index_pos[0], edge_index_neg[0]])
    edst = jnp.concatenate([edge_index_pos[1], edge_index_neg[1]])
    out = _final_kernel(s1.reshape(-1), s2.reshape(-1), esrc, edst)
    return out.reshape(EPAIR, 1)


# speedup vs baseline: 40.1804x; 40.1804x over previous
import functools

import jax
import jax.numpy as jnp
from jax import lax
from jax.experimental import pallas as pl
from jax.experimental.pallas import tpu as pltpu
from jax.experimental.pallas import tpu_sc as plsc

NN = 10000
NROW = 80
EE = 320000
EPAIR = 320000
NC, NS = 2, 16
NW = NC * NS


def _mesh():
    return plsc.VectorSubcoreMesh(core_axis_name="c", subcore_axis_name="s")


EPT_DEG = EE // NS


NPAD = NROW * 128
DPT = NPAD // NS


def _deg_body(dst_hbm, out_hbm, dstv, degl, rbuf, sbuf, parts):
    cid = lax.axis_index("c")
    sid = lax.axis_index("s")

    @pl.when(cid == 0)
    def _():
        pltpu.sync_copy(dst_hbm.at[pl.ds(sid * EPT_DEG, EPT_DEG)], dstv)

        def zero_step(i, carry):
            degl[pl.ds(i * 16, 16)] = jnp.zeros((16,), jnp.float32)
            return carry

        lax.fori_loop(0, NPAD // 16, zero_step, 0)

        ones = jnp.ones((16,), jnp.float32)

        def edge_step(i, carry):
            d = dstv[pl.ds(i * 16, 16)]
            plsc.addupdate_scatter(degl, [d], ones)
            return carry

        lax.fori_loop(0, EPT_DEG // 16, edge_step, 0)
        pltpu.sync_copy(degl, parts.at[sid])
        plsc.subcore_barrier()
        pltpu.sync_copy(parts.at[:, pl.ds(sid * DPT, DPT)], rbuf)

        def red_step(j, carry):
            v = rbuf[0, pl.ds(j * 16, 16)]
            for p in range(1, NS):
                v = v + rbuf[p, pl.ds(j * 16, 16)]
            sbuf[pl.ds(j * 16, 16)] = v
            return carry

        lax.fori_loop(0, DPT // 16, red_step, 0)
        pltpu.sync_copy(sbuf, out_hbm.at[pl.ds(sid * DPT, DPT)])


_deg_kernel = pl.kernel(
    _deg_body,
    out_type=jax.ShapeDtypeStruct((NPAD,), jnp.float32),
    mesh=_mesh(),
    compiler_params=pltpu.CompilerParams(needs_layout_passes=False, use_tc_tiling_on_sc=False),
    scratch_types=[
        pltpu.VMEM((EPT_DEG,), jnp.int32),
        pltpu.VMEM((NPAD,), jnp.float32),
        pltpu.VMEM((NS, DPT), jnp.float32),
        pltpu.VMEM((DPT,), jnp.float32),
        pltpu.VMEM_SHARED((NS, NPAD), jnp.float32),
    ],
)


EPT = EE // NW
KCH = 80
NCH = EPT // KCH
CPT = 1000
ZR = 50


NB = 8


def _make_spmm(C, dtype=jnp.float32):
    lanes = 16 if dtype == jnp.float32 else 32
    cb = C // lanes

    def body(g_hbm, s2d_hbm, d2d_hbm, out_a, out_b, sidx, didx, *scr):
        rbufs = scr[:NB]
        zbuf = scr[NB]
        acc = scr[NB + 1]
        sgs = scr[NB + 2:2 * NB + 2]
        sss = scr[2 * NB + 2:]
        cid = lax.axis_index("c")
        sid = lax.axis_index("s")
        wid = sid * NC + cid

        pltpu.sync_copy(s2d_hbm.at[wid], sidx.at[pl.ds(0, NCH)])
        pltpu.sync_copy(d2d_hbm.at[wid], didx.at[pl.ds(0, NCH)])
        for k in range(KCH // 16):
            sidx[NCH, pl.ds(k * 16, 16)] = jnp.zeros((16,), jnp.int32)

        def zero_row(i, carry):
            zbuf[lax.div(i, cb),
                 pl.ds(lax.mul(lax.rem(i, cb), lanes), lanes)] = (
                jnp.zeros((lanes,), dtype))
            return carry

        lax.fori_loop(0, ZR * cb, zero_row, 0)

        @pl.when(sid < 10)
        def _():
            for z in range(CPT // ZR):
                pltpu.sync_copy(zbuf, acc.at[pl.ds(sid * CPT + z * ZR, ZR)])

        plsc.subcore_barrier()

        def wait_g(a, j):
            pltpu.make_async_copy(g_hbm.at[sidx.at[a]], rbufs[j],
                                  sgs[j]).wait()

        def issue_s(a, j):
            pltpu.async_copy(rbufs[j], acc.at[didx.at[a]], sss[j], add=True)

        def wait_s(a, j):
            pltpu.make_async_copy(rbufs[j], acc.at[didx.at[a]],
                                  sss[j]).wait()

        def visit(a, j, jn, issue_gather=True):
            wait_s(a - (NB - 1), jn)
            if issue_gather:
                pltpu.async_copy(g_hbm.at[sidx.at[a + 1]], rbufs[jn],
                                 sgs[jn])
            wait_g(a, j)
            issue_s(a, j)

        for b in range(NB):
            pltpu.async_copy(g_hbm.at[sidx.at[b]], rbufs[b], sgs[b])
        for a in range(NB - 1):
            wait_g(a, a)
            issue_s(a, a)

        def ring_step(i, carry):
            for par in range(NB):
                a = NB * i + (NB - 1) + par
                visit(a, (NB - 1 + par) % NB, par % NB)
            return carry

        n_loops = (NCH - NB) // NB
        lax.fori_loop(0, n_loops, ring_step, 0)
        for a in range(NB - 1 + NB * n_loops, NCH - 1):
            visit(a, a % NB, (a + 1) % NB)
        visit(NCH - 1, (NCH - 1) % NB, NCH % NB, issue_gather=False)
        for a in range(NCH - NB + 1, NCH):
            wait_s(a, a % NB)
        plsc.subcore_barrier()

        @pl.when(jnp.logical_and(cid == 0, sid < 10))
        def _():
            pltpu.sync_copy(acc.at[pl.ds(sid * CPT, CPT)],
                            out_a.at[pl.ds(sid * CPT, CPT)])

        @pl.when(jnp.logical_and(cid == 1, sid < 10))
        def _():
            pltpu.sync_copy(acc.at[pl.ds(sid * CPT, CPT)],
                            out_b.at[pl.ds(sid * CPT, CPT)])

    return pl.kernel(
        body,
        out_type=[jax.ShapeDtypeStruct((NN, C), dtype)] * 2,
        mesh=_mesh(),
        compiler_params=pltpu.CompilerParams(needs_layout_passes=False, use_tc_tiling_on_sc=False),
        scratch_types=[
            pltpu.VMEM((NCH + 3, KCH), jnp.int32),
            pltpu.VMEM((NCH, KCH), jnp.int32),
        ] + [pltpu.VMEM((KCH, C), dtype)] * NB + [
            pltpu.VMEM((ZR, C), dtype),
            pltpu.VMEM_SHARED((NN, C), dtype),
        ] + [pltpu.SemaphoreType.DMA] * (2 * NB),
    )


_spmm128 = _make_spmm(128, jnp.bfloat16)
_spmm64 = _make_spmm(64, jnp.bfloat16)


EPT_F = EPAIR // NW


def _final_body(s1_hbm, s2_hbm, src_hbm, dst_hbm, out_hbm,
                s1v, s2v, siv, dov, ov, sem):
    cid = lax.axis_index("c")
    sid = lax.axis_index("s")
    wid = sid * NC + cid
    base = wid * EPT_F
    pltpu.async_copy(s1_hbm, s1v, sem)
    pltpu.async_copy(s2_hbm, s2v, sem)
    pltpu.async_copy(src_hbm.at[pl.ds(base, EPT_F)], siv, sem)
    pltpu.async_copy(dst_hbm.at[pl.ds(base, EPT_F)], dov, sem)
    pltpu.make_async_copy(s1_hbm, s1v, sem).wait()
    pltpu.make_async_copy(s2_hbm, s2v, sem).wait()
    pltpu.make_async_copy(src_hbm.at[pl.ds(base, EPT_F)], siv, sem).wait()
    pltpu.make_async_copy(dst_hbm.at[pl.ds(base, EPT_F)], dov, sem).wait()

    def edge_step(i, carry):
        s = siv[pl.ds(i * 16, 16)]
        d = dov[pl.ds(i * 16, 16)]
        ov[pl.ds(i * 16, 16)] = (plsc.load_gather(s1v, [s]) +
                                 plsc.load_gather(s2v, [d]))
        return carry

    lax.fori_loop(0, EPT_F // 16, edge_step, 0)
    pltpu.sync_copy(ov, out_hbm.at[pl.ds(base, EPT_F)])


_final_kernel = pl.kernel(
    _final_body,
    out_type=jax.ShapeDtypeStruct((EPAIR,), jnp.float32),
    mesh=_mesh(),
    compiler_params=pltpu.CompilerParams(needs_layout_passes=False, use_tc_tiling_on_sc=False),
    scratch_types=[
        pltpu.VMEM((NN,), jnp.float32),
        pltpu.VMEM((NN,), jnp.float32),
        pltpu.VMEM((EPT_F,), jnp.int32),
        pltpu.VMEM((EPT_F,), jnp.int32),
        pltpu.VMEM((EPT_F,), jnp.float32),
        pltpu.SemaphoreType.DMA,
    ],
)


BLK = 1000
GRID = NN // BLK


def _tc_a_body(x_ref, w_ref, deg_ref, g1_ref):
    dinv = lax.rsqrt(deg_ref[...] + 1.0)
    g1 = dinv * jnp.dot(x_ref[...], w_ref[...],
                        preferred_element_type=jnp.float32)
    g1_ref[...] = g1.astype(jnp.bfloat16)


def _tc_b_body(ta, tb, g1, deg, b1, w2, g2):
    dinv = lax.rsqrt(deg[...] + 1.0)
    t = (ta[...].astype(jnp.float32) + tb[...].astype(jnp.float32)
         + g1[...].astype(jnp.float32))
    h1 = jnp.maximum(dinv * t + b1[...], 0.0)
    g2f = dinv * jnp.dot(h1, w2[...], preferred_element_type=jnp.float32)
    g2[...] = g2f.astype(jnp.bfloat16)


def _tc_c_body(ta, tb, g2, deg, b2, fw1, fw2, fb, s1, s2):
    dinv = lax.rsqrt(deg[...] + 1.0)
    t = (ta[...].astype(jnp.float32) + tb[...].astype(jnp.float32)
         + g2[...].astype(jnp.float32))
    h2 = jnp.maximum(dinv * t + b2[...], 0.0)
    s1[...] = jnp.dot(h2, fw1[...], preferred_element_type=jnp.float32) + fb[...]
    s2[...] = jnp.dot(h2, fw2[...], preferred_element_type=jnp.float32)


def _row_spec(c):
    return pl.BlockSpec((BLK, c), lambda i: (i, 0))


def _full_spec(r, c):
    return pl.BlockSpec((r, c), lambda i: (0, 0))


_tc_a = pl.pallas_call(
    _tc_a_body,
    grid=(GRID,),
    in_specs=[_row_spec(128), _full_spec(128, 128), _row_spec(1)],
    out_specs=_row_spec(128),
    out_shape=jax.ShapeDtypeStruct((NN, 128), jnp.bfloat16),
)

_tc_b = pl.pallas_call(
    _tc_b_body,
    grid=(GRID,),
    in_specs=[_row_spec(128), _row_spec(128), _row_spec(128), _row_spec(1),
              _full_spec(1, 128), _full_spec(128, 64)],
    out_specs=_row_spec(64),
    out_shape=jax.ShapeDtypeStruct((NN, 64), jnp.bfloat16),
)

_tc_c = pl.pallas_call(
    _tc_c_body,
    grid=(GRID,),
    in_specs=[_row_spec(64), _row_spec(64), _row_spec(64), _row_spec(1),
              _full_spec(1, 64), _full_spec(64, 1), _full_spec(64, 1),
              _full_spec(1, 1)],
    out_specs=[_row_spec(1), _row_spec(1)],
    out_shape=[jax.ShapeDtypeStruct((NN, 1), jnp.float32)] * 2,
)


def kernel(x, edge_index, edge_index_pos, edge_index_neg,
           W1, b1, W2, b2, fc_w, fc_b):
    src = edge_index[0]
    dst = edge_index[1]

    deg_flat = _deg_kernel(dst)
    deg_col = deg_flat[:NN].reshape(NN, 1)

    src2d = src.reshape(NW, NCH, KCH)
    dst2d = dst.reshape(NW, NCH, KCH)
    g1 = _tc_a(x, W1, deg_col)
    t1a, t1b = _spmm128(g1, src2d, dst2d)
    g2 = _tc_b(t1a, t1b, g1, deg_col, b1.reshape(1, 128), W2)
    t2a, t2b = _spmm64(g2, src2d, dst2d)
    s1, s2 = _tc_c(t2a, t2b, g2, deg_col, b2.reshape(1, 64),
                   fc_w[:64], fc_w[64:], fc_b.reshape(1, 1))

    esrc = jnp.concatenate([edge_index_pos[0], edge_index_neg[0]])
    edst = jnp.concatenate([edge_index_pos[1], edge_index_neg[1]])
    out = _final_kernel(s1.reshape(-1), s2.reshape(-1), esrc, edst)
    return out.reshape(EPAIR, 1)
